# Initial kernel scaffold; baseline (speedup 1.0000x reference)
import jax
import jax.numpy as jnp
from jax.experimental import pallas as pl


def kernel(x, We1, be1, We2, be2, We3, be3, Wd1, bd1, Wd2, bd2, Wd3, bd3, codebook):
    raise NotImplementedError("write your pallas kernel here")


# speedup vs baseline: 1.1743x; 1.1743x over previous
import jax
import jax.numpy as jnp
from jax.experimental import pallas as pl
from jax.experimental.pallas import tpu as pltpu

_B, _DIN, _HID, _CODE, _K = 4096, 1024, 1024, 256, 1024
_BM = 512

_DEF = jax.lax.Precision.DEFAULT
_HI = jax.lax.Precision.HIGHEST


def _dot(a, b, prec):
    return jax.lax.dot_general(
        a, b, (((1,), (0,)), ((), ())),
        precision=prec, preferred_element_type=jnp.float32)


def _body(x_ref, We1_ref, be1_ref, We2_ref, be2_ref, We3_ref, be3_ref,
          Wd1_ref, bd1_ref, Wd2_ref, bd2_ref, Wd3_ref, bd3_ref, cb_ref,
          xt_ref, ze_ref, zq_ref):
    x = x_ref[...]
    h = jnp.maximum(_dot(x, We1_ref[...], _DEF) + be1_ref[...], 0.0)
    h = jnp.maximum(_dot(h, We2_ref[...], _DEF) + be2_ref[...], 0.0)
    z_e = _dot(h, We3_ref[...], _DEF) + be3_ref[...]
    ze_ref[...] = z_e

    cb = cb_ref[...]
    c_sqr = jnp.sum(cb * cb, axis=1)
    z_sqr = jnp.sum(z_e * z_e, axis=1, keepdims=True)
    zc = jax.lax.dot_general(
        z_e, cb, (((1,), (1,)), ((), ())),
        precision=_DEF, preferred_element_type=jnp.float32)
    dist = z_sqr + c_sqr[None, :] - 2.0 * zc

    m = jnp.min(dist, axis=1, keepdims=True)
    iota = jax.lax.broadcasted_iota(jnp.int32, dist.shape, 1)
    idx = jnp.min(jnp.where(dist == m, iota, _K), axis=1)
    onehot = (iota == idx[:, None]).astype(jnp.float32)
    codes = _dot(onehot, cb, _HI)
    zq_ref[...] = codes

    h = jnp.maximum(_dot(codes, Wd1_ref[...], _DEF) + bd1_ref[...], 0.0)
    h = jnp.maximum(_dot(h, Wd2_ref[...], _DEF) + bd2_ref[...], 0.0)
    xt_ref[...] = _dot(h, Wd3_ref[...], _DEF) + bd3_ref[...]


def _full(shape):
    return pl.BlockSpec(shape, lambda i: (0, 0))


def kernel(x, We1, be1, We2, be2, We3, be3,
           Wd1, bd1, Wd2, bd2, Wd3, bd3, codebook):
    grid = (_B // _BM,)
    out_shape = (
        jax.ShapeDtypeStruct((_B, _DIN), jnp.float32),
        jax.ShapeDtypeStruct((_B, _CODE), jnp.float32),
        jax.ShapeDtypeStruct((_B, _CODE), jnp.float32),
    )
    in_specs = [
        pl.BlockSpec((_BM, _DIN), lambda i: (i, 0)),
        _full((_DIN, _HID)), _full((1, _HID)),
        _full((_HID, _HID)), _full((1, _HID)),
        _full((_HID, _CODE)), _full((1, _CODE)),
        _full((_CODE, _HID)), _full((1, _HID)),
        _full((_HID, _HID)), _full((1, _HID)),
        _full((_HID, _DIN)), _full((1, _DIN)),
        _full((_K, _CODE)),
    ]
    out_specs = (
        pl.BlockSpec((_BM, _DIN), lambda i: (i, 0)),
        pl.BlockSpec((_BM, _CODE), lambda i: (i, 0)),
        pl.BlockSpec((_BM, _CODE), lambda i: (i, 0)),
    )
    xt, ze, zq = pl.pallas_call(
        _body,
        grid=grid,
        in_specs=in_specs,
        out_specs=out_specs,
        out_shape=out_shape,
        compiler_params=pltpu.CompilerParams(
            dimension_semantics=("arbitrary",),
        ),
    )(x, We1, be1.reshape(1, -1), We2, be2.reshape(1, -1),
      We3, be3.reshape(1, -1), Wd1, bd1.reshape(1, -1),
      Wd2, bd2.reshape(1, -1), Wd3, bd3.reshape(1, -1), codebook)
    return xt, ze, zq


# speedup vs baseline: 1.4203x; 1.2095x over previous
import jax
import jax.numpy as jnp
from jax.experimental import pallas as pl
from jax.experimental.pallas import tpu as pltpu

_B, _DIN, _HID, _CODE, _K = 4096, 1024, 1024, 256, 1024
_BM = 512

_DEF = jax.lax.Precision.DEFAULT


def _dot(a, b, prec=_DEF):
    return jax.lax.dot_general(
        a, b, (((1,), (0,)), ((), ())),
        precision=prec, preferred_element_type=jnp.float32)


def _body(x_ref, We1_ref, be1_ref, We2_ref, be2_ref, We3_ref, be3_ref,
          Wd1_ref, bd1_ref, Wd2_ref, bd2_ref, Wd3_ref, bd3_ref,
          cb_ref, cbh_ref, cbl_ref,
          xt_ref, ze_ref, zq_ref):
    x = x_ref[...]
    h = jnp.maximum(_dot(x, We1_ref[...]) + be1_ref[...], 0.0)
    h = jnp.maximum(_dot(h, We2_ref[...]) + be2_ref[...], 0.0)
    z_e = _dot(h, We3_ref[...]) + be3_ref[...]
    ze_ref[...] = z_e

    cb = cb_ref[...]
    c_sqr = jnp.sum(cb * cb, axis=1)
    z_sqr = jnp.sum(z_e * z_e, axis=1, keepdims=True)
    zc = jax.lax.dot_general(
        z_e, cbh_ref[...], (((1,), (1,)), ((), ())),
        precision=_DEF, preferred_element_type=jnp.float32)
    dist = z_sqr + c_sqr[None, :] - 2.0 * zc

    m = jnp.min(dist, axis=1, keepdims=True)
    iota = jax.lax.broadcasted_iota(jnp.int32, dist.shape, 1)
    idx = jnp.min(jnp.where(dist == m, iota, _K), axis=1)
    onehot = (iota == idx[:, None]).astype(jnp.bfloat16)
    codes_hi = _dot(onehot, cbh_ref[...])
    zq_ref[...] = codes_hi + _dot(onehot, cbl_ref[...])

    h = jnp.maximum(_dot(codes_hi.astype(jnp.bfloat16), Wd1_ref[...]) + bd1_ref[...], 0.0)
    h = jnp.maximum(_dot(h.astype(jnp.bfloat16), Wd2_ref[...]) + bd2_ref[...], 0.0)
    xt_ref[...] = _dot(h.astype(jnp.bfloat16), Wd3_ref[...]) + bd3_ref[...]


def _full(shape):
    return pl.BlockSpec(shape, lambda i: (0, 0))


def kernel(x, We1, be1, We2, be2, We3, be3,
           Wd1, bd1, Wd2, bd2, Wd3, bd3, codebook):
    grid = (_B // _BM,)
    out_shape = (
        jax.ShapeDtypeStruct((_B, _DIN), jnp.float32),
        jax.ShapeDtypeStruct((_B, _CODE), jnp.float32),
        jax.ShapeDtypeStruct((_B, _CODE), jnp.float32),
    )
    in_specs = [
        pl.BlockSpec((_BM, _DIN), lambda i: (i, 0)),
        _full((_DIN, _HID)), _full((1, _HID)),
        _full((_HID, _HID)), _full((1, _HID)),
        _full((_HID, _CODE)), _full((1, _CODE)),
        _full((_CODE, _HID)), _full((1, _HID)),
        _full((_HID, _HID)), _full((1, _HID)),
        _full((_HID, _DIN)), _full((1, _DIN)),
        _full((_K, _CODE)), _full((_K, _CODE)), _full((_K, _CODE)),
    ]
    out_specs = (
        pl.BlockSpec((_BM, _DIN), lambda i: (i, 0)),
        pl.BlockSpec((_BM, _CODE), lambda i: (i, 0)),
        pl.BlockSpec((_BM, _CODE), lambda i: (i, 0)),
    )
    bf = jnp.bfloat16
    xt, ze, zq = pl.pallas_call(
        _body,
        grid=grid,
        in_specs=in_specs,
        out_specs=out_specs,
        out_shape=out_shape,
        compiler_params=pltpu.CompilerParams(
            dimension_semantics=("arbitrary",),
        ),
    )(x, We1.astype(bf), be1.reshape(1, -1),
      We2.astype(bf), be2.reshape(1, -1),
      We3.astype(bf), be3.reshape(1, -1),
      Wd1.astype(bf), bd1.reshape(1, -1),
      Wd2.astype(bf), bd2.reshape(1, -1),
      Wd3.astype(bf), bd3.reshape(1, -1),
      codebook, codebook.astype(bf),
      (codebook - codebook.astype(bf).astype(jnp.float32)).astype(bf))
    return xt, ze, zq


# speedup vs baseline: 1.6704x; 1.1761x over previous
import jax
import jax.numpy as jnp
from jax.experimental import pallas as pl
from jax.experimental.pallas import tpu as pltpu

_B, _DIN, _HID, _CODE, _K = 4096, 1024, 1024, 256, 1024
_BM = 512

_DEF = jax.lax.Precision.DEFAULT


def _dot(a, b, prec=_DEF):
    return jax.lax.dot_general(
        a, b, (((1,), (0,)), ((), ())),
        precision=prec, preferred_element_type=jnp.float32)


def _body(x_ref, We1_ref, be1_ref, We2_ref, be2_ref, We3_ref, be3_ref,
          Wd1_ref, bd1_ref, Wd2_ref, bd2_ref, Wd3_ref, bd3_ref, cb_ref,
          xt_ref, ze_ref, zq_ref,
          we1_s, we2_s, we3_s, wd1_s, wd2_s, wd3_s, cbh_s, cbl_s, csq_s):
    bf = jnp.bfloat16

    @pl.when(pl.program_id(0) == 0)
    def _init():
        we1_s[...] = We1_ref[...].astype(bf)
        we2_s[...] = We2_ref[...].astype(bf)
        we3_s[...] = We3_ref[...].astype(bf)
        wd1_s[...] = Wd1_ref[...].astype(bf)
        wd2_s[...] = Wd2_ref[...].astype(bf)
        wd3_s[...] = Wd3_ref[...].astype(bf)
        cbf = cb_ref[...]
        hi = cbf.astype(bf)
        cbh_s[...] = hi
        cbl_s[...] = (cbf - hi.astype(jnp.float32)).astype(bf)
        csq_s[...] = jnp.sum(cbf * cbf, axis=1)[None, :]

    x = x_ref[...]
    h = jnp.maximum(_dot(x, we1_s[...]) + be1_ref[...], 0.0)
    h = jnp.maximum(_dot(h, we2_s[...]) + be2_ref[...], 0.0)
    z_e = _dot(h, we3_s[...]) + be3_ref[...]
    ze_ref[...] = z_e

    z_sqr = jnp.sum(z_e * z_e, axis=1, keepdims=True)
    zc = jax.lax.dot_general(
        z_e, cbh_s[...], (((1,), (1,)), ((), ())),
        precision=_DEF, preferred_element_type=jnp.float32)
    dist = z_sqr + csq_s[...] - 2.0 * zc

    m = jnp.min(dist, axis=1, keepdims=True)
    iota = jax.lax.broadcasted_iota(jnp.int32, dist.shape, 1)
    idx = jnp.min(jnp.where(dist == m, iota, _K), axis=1)
    onehot = (iota == idx[:, None]).astype(bf)
    codes_hi = _dot(onehot, cbh_s[...])
    zq_ref[...] = codes_hi + _dot(onehot, cbl_s[...])

    h = jnp.maximum(_dot(codes_hi.astype(bf), wd1_s[...]) + bd1_ref[...], 0.0)
    h = jnp.maximum(_dot(h.astype(bf), wd2_s[...]) + bd2_ref[...], 0.0)
    xt_ref[...] = _dot(h.astype(bf), wd3_s[...]) + bd3_ref[...]


def _full(shape):
    return pl.BlockSpec(shape, lambda i: (0, 0))


def kernel(x, We1, be1, We2, be2, We3, be3,
           Wd1, bd1, Wd2, bd2, Wd3, bd3, codebook):
    grid = (_B // _BM,)
    out_shape = (
        jax.ShapeDtypeStruct((_B, _DIN), jnp.float32),
        jax.ShapeDtypeStruct((_B, _CODE), jnp.float32),
        jax.ShapeDtypeStruct((_B, _CODE), jnp.float32),
    )
    in_specs = [
        pl.BlockSpec((_BM, _DIN), lambda i: (i, 0)),
        _full((_DIN, _HID)), _full((1, _HID)),
        _full((_HID, _HID)), _full((1, _HID)),
        _full((_HID, _CODE)), _full((1, _CODE)),
        _full((_CODE, _HID)), _full((1, _HID)),
        _full((_HID, _HID)), _full((1, _HID)),
        _full((_HID, _DIN)), _full((1, _DIN)),
        _full((_K, _CODE)),
    ]
    out_specs = (
        pl.BlockSpec((_BM, _DIN), lambda i: (i, 0)),
        pl.BlockSpec((_BM, _CODE), lambda i: (i, 0)),
        pl.BlockSpec((_BM, _CODE), lambda i: (i, 0)),
    )
    bf = jnp.bfloat16
    scratch_shapes = [
        pltpu.VMEM((_DIN, _HID), bf),
        pltpu.VMEM((_HID, _HID), bf),
        pltpu.VMEM((_HID, _CODE), bf),
        pltpu.VMEM((_CODE, _HID), bf),
        pltpu.VMEM((_HID, _HID), bf),
        pltpu.VMEM((_HID, _DIN), bf),
        pltpu.VMEM((_K, _CODE), bf),
        pltpu.VMEM((_K, _CODE), bf),
        pltpu.VMEM((1, _K), jnp.float32),
    ]
    xt, ze, zq = pl.pallas_call(
        _body,
        grid=grid,
        in_specs=in_specs,
        out_specs=out_specs,
        out_shape=out_shape,
        scratch_shapes=scratch_shapes,
        compiler_params=pltpu.CompilerParams(
            dimension_semantics=("arbitrary",),
        ),
    )(x, We1, be1.reshape(1, -1), We2, be2.reshape(1, -1),
      We3, be3.reshape(1, -1), Wd1, bd1.reshape(1, -1),
      Wd2, bd2.reshape(1, -1), Wd3, bd3.reshape(1, -1), codebook)
    return xt, ze, zq
